# Initial kernel scaffold; baseline (speedup 1.0000x reference)
import jax
import jax.numpy as jnp
from jax.experimental import pallas as pl


def kernel(x, adj, W1_rel, W1_root, b1, W2_rel, W2_root, b2, bn1_w, bn1_b, bn2_w, bn2_b):
    raise NotImplementedError("write your pallas kernel here")


# speedup vs baseline: 1.8957x; 1.8957x over previous
import functools

import jax
import jax.numpy as jnp
from jax.experimental import pallas as pl
from jax.experimental.pallas import tpu as pltpu


def _gnn_kernel(B, N, HID, OUT,
                x_ref, adj_ref, u_ref, v_ref, w_ref, W2r_ref, W2o_ref,
                b2_ref, g1w_ref, g1b_ref, g2w_ref, g2b_ref,
                out_ref,
                sA, sS, sC, sm1, sq1, sg1, sk1, h2s, sm2, sq2, sg2, sk2):
    p = pl.program_id(0)
    b = pl.program_id(1)

    @pl.when(p == 0)
    def _phase0():
        adj_b = adj_ref[0]
        xb = x_ref[0]
        m1 = jnp.dot(adj_b, xb, preferred_element_type=jnp.float32)
        deg = jnp.maximum(jnp.sum(adj_b, axis=1, keepdims=True), 1.0)
        a = m1 / deg

        u = u_ref[...]
        v = v_ref[...]
        w = w_ref[...]
        Suu = jnp.sum(u * u)
        Svv = jnp.sum(v * v)
        Sww = jnp.sum(w * w)
        Suv = jnp.sum(u * v)
        Suw = jnp.sum(u * w)
        Svw = jnp.sum(v * w)

        q = (a * a * Suu + xb * xb * Svv + Sww
             + 2.0 * a * xb * Suv + 2.0 * a * Suw + 2.0 * xb * Svw)
        nrm = jnp.maximum(jnp.sqrt(q), 1e-12)
        A = a / nrm
        S = xb / nrm
        C = 1.0 / nrm
        sA[b] = A
        sS[b] = S
        sC[b] = C

        Su = jnp.sum(u)
        Sv = jnp.sum(v)
        Sw = jnp.sum(w)
        rowsum = A * Su + S * Sv + C * Sw
        rowsq = q / (nrm * nrm)

        @pl.when(b == 0)
        def _():
            sm1[...] = rowsum
            sq1[...] = rowsq

        @pl.when(b != 0)
        def _():
            sm1[...] += rowsum
            sq1[...] += rowsq

    @pl.when(p == 1)
    def _phase1():
        @pl.when(b == 0)
        def _():
            mean = sm1[...] / (B * HID)
            ex2 = sq1[...] / (B * HID)
            var = ex2 - mean * mean
            inv = jax.lax.rsqrt(var + 1e-5)
            g = inv * g1w_ref[...]
            sg1[...] = g
            sk1[...] = g1b_ref[...] - mean * g

        A = sA[b]
        S = sS[b]
        C = sC[b]
        h1n = A * u_ref[...] + S * v_ref[...] + C * w_ref[...]
        hb = jnp.maximum(h1n * sg1[...] + sk1[...], 0.0)

        adj_b = adj_ref[0]
        deg = jnp.maximum(jnp.sum(adj_b, axis=1, keepdims=True), 1.0)
        m2 = jnp.dot(adj_b, hb, preferred_element_type=jnp.float32)
        t = m2 / deg
        h2 = (jnp.dot(t, W2r_ref[...], preferred_element_type=jnp.float32)
              + jnp.dot(hb, W2o_ref[...], preferred_element_type=jnp.float32)
              + b2_ref[...])
        q2 = jnp.sum(h2 * h2, axis=1, keepdims=True)
        nrm2 = jnp.maximum(jnp.sqrt(q2), 1e-12)
        h2n = h2 / nrm2
        h2s[b] = h2n

        rowsum = jnp.sum(h2n, axis=1, keepdims=True)
        rowsq = q2 / (nrm2 * nrm2)

        @pl.when(b == 0)
        def _():
            sm2[...] = rowsum
            sq2[...] = rowsq

        @pl.when(b != 0)
        def _():
            sm2[...] += rowsum
            sq2[...] += rowsq

    @pl.when(p == 2)
    def _phase2():
        @pl.when(b == 0)
        def _():
            mean = sm2[...] / (B * OUT)
            ex2 = sq2[...] / (B * OUT)
            var = ex2 - mean * mean
            inv = jax.lax.rsqrt(var + 1e-5)
            g = inv * g2w_ref[...]
            sg2[...] = g
            sk2[...] = g2b_ref[...] - mean * g

        out_ref[0] = jnp.maximum(h2s[b] * sg2[...] + sk2[...], 0.0)


def kernel(x, adj, W1_rel, W1_root, b1, W2_rel, W2_root, b2,
           bn1_w, bn1_b, bn2_w, bn2_b):
    B, N, _ = x.shape
    HID = W1_rel.shape[1]
    OUT = W2_rel.shape[1]

    u = W1_rel.reshape(1, HID).astype(jnp.float32)
    v = W1_root.reshape(1, HID).astype(jnp.float32)
    w = b1.reshape(1, HID).astype(jnp.float32)
    b2r = b2.reshape(1, OUT).astype(jnp.float32)
    g1w = bn1_w.reshape(N, 1).astype(jnp.float32)
    g1b = bn1_b.reshape(N, 1).astype(jnp.float32)
    g2w = bn2_w.reshape(N, 1).astype(jnp.float32)
    g2b = bn2_b.reshape(N, 1).astype(jnp.float32)

    grid = (3, B)

    def const_spec(shape):
        nd = len(shape)
        return pl.BlockSpec(shape, lambda p, b, _nd=nd: (0,) * _nd)

    in_specs = [
        pl.BlockSpec((1, N, 1), lambda p, b: (jnp.where(p == 0, b, B - 1), 0, 0)),
        pl.BlockSpec((1, N, N), lambda p, b: (jnp.where(p < 2, b, B - 1), 0, 0)),
        const_spec((1, HID)),
        const_spec((1, HID)),
        const_spec((1, HID)),
        const_spec((HID, OUT)),
        const_spec((HID, OUT)),
        const_spec((1, OUT)),
        const_spec((N, 1)),
        const_spec((N, 1)),
        const_spec((N, 1)),
        const_spec((N, 1)),
    ]
    out_spec = pl.BlockSpec((1, N, OUT),
                            lambda p, b: (jnp.where(p == 2, b, 0), 0, 0))

    scratch_shapes = [
        pltpu.VMEM((B, N, 1), jnp.float32),
        pltpu.VMEM((B, N, 1), jnp.float32),
        pltpu.VMEM((B, N, 1), jnp.float32),
        pltpu.VMEM((N, 1), jnp.float32),
        pltpu.VMEM((N, 1), jnp.float32),
        pltpu.VMEM((N, 1), jnp.float32),
        pltpu.VMEM((N, 1), jnp.float32),
        pltpu.VMEM((B, N, OUT), jnp.float32),
        pltpu.VMEM((N, 1), jnp.float32),
        pltpu.VMEM((N, 1), jnp.float32),
        pltpu.VMEM((N, 1), jnp.float32),
        pltpu.VMEM((N, 1), jnp.float32),
    ]

    fn = functools.partial(_gnn_kernel, B, N, HID, OUT)
    return pl.pallas_call(
        fn,
        grid=grid,
        in_specs=in_specs,
        out_specs=out_spec,
        out_shape=jax.ShapeDtypeStruct((B, N, OUT), jnp.float32),
        scratch_shapes=scratch_shapes,
        compiler_params=pltpu.CompilerParams(
            vmem_limit_bytes=100 * 1024 * 1024,
        ),
    )(x.astype(jnp.float32), adj.astype(jnp.float32), u, v, w,
      W2_rel.astype(jnp.float32), W2_root.astype(jnp.float32), b2r,
      g1w, g1b, g2w, g2b)


# speedup vs baseline: 1.9117x; 1.0084x over previous
import functools

import jax
import jax.numpy as jnp
from jax.experimental import pallas as pl
from jax.experimental.pallas import tpu as pltpu


def _gnn_kernel(B, N, HID, OUT,
                x_ref, adj_ref, u_ref, v_ref, w_ref, W2r_ref, W2o_ref,
                b2_ref, g1w_ref, g1b_ref, g2w_ref, g2b_ref,
                out_ref,
                sA, sS, sC, sD, adjs, sm1, sq1, sg1, sk1, h2s,
                sm2, sq2, sg2, sk2):
    p = pl.program_id(0)
    b = pl.program_id(1)

    @pl.when(p == 0)
    def _phase0():
        adj_b = adj_ref[0]
        adjs[b] = adj_b
        xb = x_ref[0]
        m1 = jnp.dot(adj_b, xb, preferred_element_type=jnp.float32)
        deg = jnp.maximum(jnp.sum(adj_b, axis=1, keepdims=True), 1.0)
        sD[b] = deg
        a = m1 / deg

        u = u_ref[...]
        v = v_ref[...]
        w = w_ref[...]
        Suu = jnp.sum(u * u)
        Svv = jnp.sum(v * v)
        Sww = jnp.sum(w * w)
        Suv = jnp.sum(u * v)
        Suw = jnp.sum(u * w)
        Svw = jnp.sum(v * w)

        q = (a * a * Suu + xb * xb * Svv + Sww
             + 2.0 * a * xb * Suv + 2.0 * a * Suw + 2.0 * xb * Svw)
        nrm = jnp.maximum(jnp.sqrt(q), 1e-12)
        A = a / nrm
        S = xb / nrm
        C = 1.0 / nrm
        sA[b] = A
        sS[b] = S
        sC[b] = C

        Su = jnp.sum(u)
        Sv = jnp.sum(v)
        Sw = jnp.sum(w)
        rowsum = A * Su + S * Sv + C * Sw
        rowsq = q / (nrm * nrm)

        @pl.when(b == 0)
        def _():
            sm1[...] = rowsum
            sq1[...] = rowsq

        @pl.when(b != 0)
        def _():
            sm1[...] += rowsum
            sq1[...] += rowsq

    @pl.when(p == 1)
    def _phase1():
        @pl.when(b == 0)
        def _():
            mean = sm1[...] / (B * HID)
            ex2 = sq1[...] / (B * HID)
            var = ex2 - mean * mean
            inv = jax.lax.rsqrt(var + 1e-5)
            g = inv * g1w_ref[...]
            sg1[...] = g
            sk1[...] = g1b_ref[...] - mean * g

        A = sA[b]
        S = sS[b]
        C = sC[b]
        h1n = A * u_ref[...] + S * v_ref[...] + C * w_ref[...]
        hb = jnp.maximum(h1n * sg1[...] + sk1[...], 0.0)

        adj_b = adjs[b]
        deg = sD[b]
        m2 = jnp.dot(adj_b, hb, preferred_element_type=jnp.float32)
        t = m2 / deg
        h2 = (jnp.dot(t, W2r_ref[...], preferred_element_type=jnp.float32)
              + jnp.dot(hb, W2o_ref[...], preferred_element_type=jnp.float32)
              + b2_ref[...])
        q2 = jnp.sum(h2 * h2, axis=1, keepdims=True)
        nrm2 = jnp.maximum(jnp.sqrt(q2), 1e-12)
        h2n = h2 / nrm2
        h2s[b] = h2n

        rowsum = jnp.sum(h2n, axis=1, keepdims=True)
        rowsq = q2 / (nrm2 * nrm2)

        @pl.when(b == 0)
        def _():
            sm2[...] = rowsum
            sq2[...] = rowsq

        @pl.when(b != 0)
        def _():
            sm2[...] += rowsum
            sq2[...] += rowsq

    @pl.when(p == 2)
    def _phase2():
        @pl.when(b == 0)
        def _():
            mean = sm2[...] / (B * OUT)
            ex2 = sq2[...] / (B * OUT)
            var = ex2 - mean * mean
            inv = jax.lax.rsqrt(var + 1e-5)
            g = inv * g2w_ref[...]
            sg2[...] = g
            sk2[...] = g2b_ref[...] - mean * g

        out_ref[0] = jnp.maximum(h2s[b] * sg2[...] + sk2[...], 0.0)


def kernel(x, adj, W1_rel, W1_root, b1, W2_rel, W2_root, b2,
           bn1_w, bn1_b, bn2_w, bn2_b):
    B, N, _ = x.shape
    HID = W1_rel.shape[1]
    OUT = W2_rel.shape[1]

    u = W1_rel.reshape(1, HID).astype(jnp.float32)
    v = W1_root.reshape(1, HID).astype(jnp.float32)
    w = b1.reshape(1, HID).astype(jnp.float32)
    b2r = b2.reshape(1, OUT).astype(jnp.float32)
    g1w = bn1_w.reshape(N, 1).astype(jnp.float32)
    g1b = bn1_b.reshape(N, 1).astype(jnp.float32)
    g2w = bn2_w.reshape(N, 1).astype(jnp.float32)
    g2b = bn2_b.reshape(N, 1).astype(jnp.float32)

    grid = (3, B)

    def const_spec(shape):
        nd = len(shape)
        return pl.BlockSpec(shape, lambda p, b, _nd=nd: (0,) * _nd)

    in_specs = [
        pl.BlockSpec((1, N, 1), lambda p, b: (jnp.where(p == 0, b, B - 1), 0, 0)),
        pl.BlockSpec((1, N, N), lambda p, b: (jnp.where(p == 0, b, B - 1), 0, 0)),
        const_spec((1, HID)),
        const_spec((1, HID)),
        const_spec((1, HID)),
        const_spec((HID, OUT)),
        const_spec((HID, OUT)),
        const_spec((1, OUT)),
        const_spec((N, 1)),
        const_spec((N, 1)),
        const_spec((N, 1)),
        const_spec((N, 1)),
    ]
    out_spec = pl.BlockSpec((1, N, OUT),
                            lambda p, b: (jnp.where(p == 2, b, 0), 0, 0))

    scratch_shapes = [
        pltpu.VMEM((B, N, 1), jnp.float32),
        pltpu.VMEM((B, N, 1), jnp.float32),
        pltpu.VMEM((B, N, 1), jnp.float32),
        pltpu.VMEM((B, N, 1), jnp.float32),
        pltpu.VMEM((B, N, N), jnp.float32),
        pltpu.VMEM((N, 1), jnp.float32),
        pltpu.VMEM((N, 1), jnp.float32),
        pltpu.VMEM((N, 1), jnp.float32),
        pltpu.VMEM((N, 1), jnp.float32),
        pltpu.VMEM((B, N, OUT), jnp.float32),
        pltpu.VMEM((N, 1), jnp.float32),
        pltpu.VMEM((N, 1), jnp.float32),
        pltpu.VMEM((N, 1), jnp.float32),
        pltpu.VMEM((N, 1), jnp.float32),
    ]

    fn = functools.partial(_gnn_kernel, B, N, HID, OUT)
    return pl.pallas_call(
        fn,
        grid=grid,
        in_specs=in_specs,
        out_specs=out_spec,
        out_shape=jax.ShapeDtypeStruct((B, N, OUT), jnp.float32),
        scratch_shapes=scratch_shapes,
        compiler_params=pltpu.CompilerParams(
            vmem_limit_bytes=100 * 1024 * 1024,
        ),
    )(x.astype(jnp.float32), adj.astype(jnp.float32), u, v, w,
      W2_rel.astype(jnp.float32), W2_root.astype(jnp.float32), b2r,
      g1w, g1b, g2w, g2b)
